# Initial kernel scaffold; baseline (speedup 1.0000x reference)
import jax
import jax.numpy as jnp
from jax.experimental import pallas as pl


def kernel(x, edge_index, edge_attr, batch, mlp_W1, mlp_b1, mlp_W2, mlp_b2, eps, gamma, beta, pred_W, pred_b):
    raise NotImplementedError("write your pallas kernel here")


# speedup vs baseline: 2.9456x; 2.9456x over previous
import functools

import jax
import jax.numpy as jnp
from jax import lax
from jax.experimental import pallas as pl
from jax.experimental.pallas import tpu as pltpu
from jax.experimental.pallas import tpu_sc as plsc

N = 10000
E = 320000
D = 128
G = 256

NC = 2
NS = 16
NW = NC * NS
EPT = E // NW
C = 80
NCHUNK = EPT // C
NRC = N // C

_sc_mesh = plsc.VectorSubcoreMesh(
    core_axis_name="c", subcore_axis_name="s", num_cores=NC, num_subcores=NS)


@functools.partial(
    pl.kernel,
    out_type=jax.ShapeDtypeStruct((NC, N, D), jnp.float32),
    mesh=_sc_mesh,
    scratch_types=[
        pltpu.VMEM((C,), jnp.int32),
        pltpu.VMEM((C,), jnp.int32),
        pltpu.VMEM((C, D), jnp.float32),
        pltpu.VMEM((C, D), jnp.float32),
        pltpu.VMEM_SHARED((N, D), jnp.float32),
        pltpu.SemaphoreType.DMA,
    ],
)
def _gin_agg(h_hbm, src_hbm, dst_hbm, ea_hbm, out_hbm,
             src_v, dst_v, rows_v, ea_v, agg_sh, sem):
    c = lax.axis_index("c")
    s = lax.axis_index("s")
    wid = s * NC + c

    zv = jnp.zeros((16,), jnp.float32)

    def zfill(j, carry):
        for q in range(D // 16):
            ea_v[j, pl.ds(q * 16, 16)] = zv
        return carry

    lax.fori_loop(0, C, zfill, 0)
    for i in range((NRC + NS - 1) // NS):
        idx = s + i * NS

        @pl.when(idx < NRC)
        def _():
            pltpu.sync_copy(ea_v, agg_sh.at[pl.ds(idx * C, C)])

    plsc.subcore_barrier()

    def chunk(k, carry):
        base = wid * EPT + k * C
        pltpu.sync_copy(src_hbm.at[pl.ds(base, C)], src_v)
        pltpu.sync_copy(dst_hbm.at[pl.ds(base, C)], dst_v)
        pltpu.async_copy(h_hbm.at[src_v], rows_v, sem).wait()
        pltpu.sync_copy(ea_hbm.at[pl.ds(base, C)], ea_v)

        def cb(j, icarry):
            for q in range(D // 16):
                sl = pl.ds(q * 16, 16)
                rows_v[j, sl] = jnp.maximum(rows_v[j, sl] + ea_v[j, sl], 0.0)
            return icarry

        lax.fori_loop(0, C, cb, 0)
        pltpu.sync_copy(rows_v, agg_sh.at[dst_v], add=True)
        return carry

    lax.fori_loop(0, NCHUNK, chunk, 0)

    plsc.subcore_barrier()
    for i in range((NRC + NS - 1) // NS):
        idx = s + i * NS

        @pl.when(idx < NRC)
        def _():
            pltpu.sync_copy(agg_sh.at[pl.ds(idx * C, C)],
                            out_hbm.at[c, pl.ds(idx * C, C)])


MLP_R = 1000


def _mlp_body(scale_ref, h_ref, a0_ref, a1_ref, w1_ref, b1_ref, w2_ref,
              b2_ref, g_ref, bt_ref, o_ref, *, relu_out):
    z = h_ref[...] * scale_ref[...] + a0_ref[...] + a1_ref[...]
    z = jnp.dot(z, w1_ref[...], preferred_element_type=jnp.float32) + b1_ref[...]
    z = jnp.maximum(z, 0.0)
    z = jnp.dot(z, w2_ref[...], preferred_element_type=jnp.float32) + b2_ref[...]
    z = z * g_ref[...] + bt_ref[...]
    if relu_out:
        z = jnp.maximum(z, 0.0)
    o_ref[...] = z


def _mlp_layer(scale, h, a0, a1, w1, b1, w2, b2, gamma, beta, relu_out):
    grid = (N // MLP_R,)
    row = lambda i: (i, 0)
    full = lambda i: (0, 0)
    return pl.pallas_call(
        functools.partial(_mlp_body, relu_out=relu_out),
        grid=grid,
        in_specs=[
            pl.BlockSpec((1, D), full),
            pl.BlockSpec((MLP_R, D), row),
            pl.BlockSpec((MLP_R, D), row),
            pl.BlockSpec((MLP_R, D), row),
            pl.BlockSpec((D, 2 * D), full),
            pl.BlockSpec((1, 2 * D), full),
            pl.BlockSpec((2 * D, D), full),
            pl.BlockSpec((1, D), full),
            pl.BlockSpec((1, D), full),
            pl.BlockSpec((1, D), full),
        ],
        out_specs=pl.BlockSpec((MLP_R, D), row),
        out_shape=jax.ShapeDtypeStruct((N, D), jnp.float32),
    )(scale, h, a0, a1, w1, b1, w2, b2, gamma, beta)


POOL_R = 1000
_POOL_NB = N // POOL_R
T_OUT = 128


def _pool_body(batch_ref, h_ref, pw_ref, pb_ref, o_ref, acc_ref, cnt_ref):
    i = pl.program_id(0)

    @pl.when(i == 0)
    def _():
        acc_ref[...] = jnp.zeros_like(acc_ref)
        cnt_ref[...] = jnp.zeros_like(cnt_ref)

    bb = batch_ref[...]
    iot = lax.broadcasted_iota(jnp.int32, (POOL_R, G), 1)
    oh = (iot == bb).astype(jnp.float32)
    acc_ref[...] += lax.dot_general(
        oh, h_ref[...], (((0,), (0,)), ((), ())),
        preferred_element_type=jnp.float32)
    cnt_ref[...] += lax.dot_general(
        oh, jnp.ones((POOL_R, 1), jnp.float32), (((0,), (0,)), ((), ())),
        preferred_element_type=jnp.float32)

    @pl.when(i == _POOL_NB - 1)
    def _():
        hg = acc_ref[...] / jnp.maximum(cnt_ref[...], 1.0)
        o_ref[...] = jnp.dot(hg, pw_ref[...],
                             preferred_element_type=jnp.float32) + pb_ref[...]


def _pool_head(batch_f, h, pred_W, pred_b):
    row = lambda i: (i, 0)
    full = lambda i: (0, 0)
    return pl.pallas_call(
        _pool_body,
        grid=(_POOL_NB,),
        in_specs=[
            pl.BlockSpec((POOL_R, 1), row),
            pl.BlockSpec((POOL_R, D), row),
            pl.BlockSpec((D, T_OUT), full),
            pl.BlockSpec((1, T_OUT), full),
        ],
        out_specs=pl.BlockSpec((G, T_OUT), full),
        out_shape=jax.ShapeDtypeStruct((G, T_OUT), jnp.float32),
        scratch_shapes=[
            pltpu.VMEM((G, D), jnp.float32),
            pltpu.VMEM((G, 1), jnp.float32),
        ],
    )(batch_f, h, pred_W, pred_b)


def kernel(x, edge_index, edge_attr, batch, mlp_W1, mlp_b1, mlp_W2, mlp_b2,
           eps, gamma, beta, pred_W, pred_b):
    src = edge_index[0]
    dst = edge_index[1]
    h = x
    num_layers = mlp_W1.shape[0]
    for i in range(num_layers):
        agg = _gin_agg(h, src, dst, edge_attr)
        scale = jnp.full((1, D), 1.0, jnp.float32) * (1.0 + eps[i])
        h = _mlp_layer(scale, h, agg[0], agg[1], mlp_W1[i],
                       mlp_b1[i].reshape(1, -1), mlp_W2[i],
                       mlp_b2[i].reshape(1, -1), gamma[i].reshape(1, -1),
                       beta[i].reshape(1, -1), relu_out=(i != num_layers - 1))
    return _pool_head(batch.reshape(N, 1), h, pred_W, pred_b.reshape(1, -1))


# speedup vs baseline: 7.3953x; 2.5106x over previous
import functools

import jax
import jax.numpy as jnp
from jax import lax
from jax.experimental import pallas as pl
from jax.experimental.pallas import tpu as pltpu
from jax.experimental.pallas import tpu_sc as plsc

N = 10000
E = 320000
D = 128
G = 256

NC = 2
NS = 16
NW = NC * NS
EPT = E // NW
C = 80
NCHUNK = EPT // C
NRC = N // C

_sc_mesh = plsc.VectorSubcoreMesh(
    core_axis_name="c", subcore_axis_name="s", num_cores=NC, num_subcores=NS)


@functools.partial(
    pl.kernel,
    out_type=jax.ShapeDtypeStruct((NC, N, D), jnp.float32),
    mesh=_sc_mesh,
    scratch_types=[
        pltpu.VMEM((2, C), jnp.int32),
        pltpu.VMEM((4, C), jnp.int32),
        pltpu.VMEM((2, C, D), jnp.float32),
        pltpu.VMEM((2, C, D), jnp.float32),
        pltpu.VMEM_SHARED((N, D), jnp.float32),
        pltpu.SemaphoreType.DMA((2,)),
        pltpu.SemaphoreType.DMA((4,)),
        pltpu.SemaphoreType.DMA((2,)),
        pltpu.SemaphoreType.DMA((2,)),
        pltpu.SemaphoreType.DMA((2,)),
    ],
)
def _gin_agg(h_hbm, src_hbm, dst_hbm, ea_hbm, out_hbm,
             src_v, dst_v, rows_v, ea_v, agg_sh,
             sem_src, sem_dst, sem_g, sem_e, sem_s):
    c = lax.axis_index("c")
    s = lax.axis_index("s")
    wid = s * NC + c
    ebase = wid * EPT

    zv = jnp.zeros((16,), jnp.float32)

    def zfill(j, carry):
        for q in range(D // 16):
            ea_v[0, j, pl.ds(q * 16, 16)] = zv
        return carry

    lax.fori_loop(0, C, zfill, 0)
    for i in range((NRC + NS - 1) // NS):
        idx = s + i * NS

        @pl.when(idx < NRC)
        def _():
            pltpu.sync_copy(ea_v.at[0], agg_sh.at[pl.ds(idx * C, C)])

    plsc.subcore_barrier()

    def issue_src(k, b):
        pltpu.async_copy(src_hbm.at[pl.ds(ebase + k * C, C)], src_v.at[b],
                         sem_src.at[b])

    def wait_src(k, b):
        pltpu.make_async_copy(src_hbm.at[pl.ds(ebase + k * C, C)],
                              src_v.at[b], sem_src.at[b]).wait()

    def issue_dst(k, b):
        pltpu.async_copy(dst_hbm.at[pl.ds(ebase + k * C, C)], dst_v.at[b],
                         sem_dst.at[b])

    def wait_dst(k, b):
        pltpu.make_async_copy(dst_hbm.at[pl.ds(ebase + k * C, C)],
                              dst_v.at[b], sem_dst.at[b]).wait()

    def issue_body(b):
        pltpu.async_copy(h_hbm.at[src_v.at[b]], rows_v.at[b], sem_g.at[b])

    def issue_ea(k, b):
        pltpu.async_copy(ea_hbm.at[pl.ds(ebase + k * C, C)], ea_v.at[b],
                         sem_e.at[b])

    def wait_body(k, b):
        pltpu.make_async_copy(h_hbm.at[src_v.at[b]], rows_v.at[b],
                              sem_g.at[b]).wait()
        pltpu.make_async_copy(ea_hbm.at[pl.ds(ebase + k * C, C)],
                              ea_v.at[b], sem_e.at[b]).wait()

    def wait_scatter(bs, bd):
        pltpu.make_async_copy(rows_v.at[bs], agg_sh.at[dst_v.at[bd]],
                              sem_s.at[bs]).wait()

    def compute(bs):
        @plsc.parallel_loop(0, C)
        def _(j):
            for q in range(D // 16):
                sl = pl.ds(q * 16, 16)
                rows_v[bs, j, sl] = jnp.maximum(
                    rows_v[bs, j, sl] + ea_v[bs, j, sl], 0.0)

    issue_src(0, 0)
    issue_dst(0, 0)
    issue_src(1, 1)
    issue_dst(1, 1)
    wait_src(0, 0)
    issue_body(0)
    issue_ea(0, 0)

    def pipeline_step(k, b4):
        bs = b4 % 2
        bo = 1 - bs
        pbd = (b4 - 1) % 4

        @pl.when(k >= 1)
        def _():
            wait_scatter(bo, pbd)

        wait_src(k + 1, bo)
        issue_body(bo)
        issue_ea(k + 1, bo)
        wait_body(k, bs)
        compute(bs)
        wait_dst(k, b4)
        pltpu.async_copy(rows_v.at[bs], agg_sh.at[dst_v.at[b4]],
                         sem_s.at[bs], add=True)

        @pl.when(k + 2 < NCHUNK)
        def _():
            issue_src(k + 2, bs)
            issue_dst(k + 2, (b4 + 2) % 4)

    def quad(p, carry):
        for b4 in range(4):
            pipeline_step(4 * p + b4, b4)
        return carry

    lax.fori_loop(0, (NCHUNK - 1) // 4, quad, 0)

    kl = NCHUNK - 1
    wait_scatter(1, 3)
    wait_body(kl, 0)
    compute(0)
    wait_dst(kl, 0)
    pltpu.sync_copy(rows_v.at[0], agg_sh.at[dst_v.at[0]], add=True)

    plsc.subcore_barrier()
    for i in range((NRC + NS - 1) // NS):
        idx = s + i * NS

        @pl.when(idx < NRC)
        def _():
            pltpu.sync_copy(agg_sh.at[pl.ds(idx * C, C)],
                            out_hbm.at[c, pl.ds(idx * C, C)])


MLP_R = 1000


def _mlp_body(scale_ref, h_ref, a0_ref, a1_ref, w1_ref, b1_ref, w2_ref,
              b2_ref, g_ref, bt_ref, o_ref, *, relu_out):
    z = h_ref[...] * scale_ref[...] + a0_ref[...] + a1_ref[...]
    z = jnp.dot(z, w1_ref[...], preferred_element_type=jnp.float32) + b1_ref[...]
    z = jnp.maximum(z, 0.0)
    z = jnp.dot(z, w2_ref[...], preferred_element_type=jnp.float32) + b2_ref[...]
    z = z * g_ref[...] + bt_ref[...]
    if relu_out:
        z = jnp.maximum(z, 0.0)
    o_ref[...] = z


def _mlp_layer(scale, h, a0, a1, w1, b1, w2, b2, gamma, beta, relu_out):
    grid = (N // MLP_R,)
    row = lambda i: (i, 0)
    full = lambda i: (0, 0)
    return pl.pallas_call(
        functools.partial(_mlp_body, relu_out=relu_out),
        grid=grid,
        in_specs=[
            pl.BlockSpec((1, D), full),
            pl.BlockSpec((MLP_R, D), row),
            pl.BlockSpec((MLP_R, D), row),
            pl.BlockSpec((MLP_R, D), row),
            pl.BlockSpec((D, 2 * D), full),
            pl.BlockSpec((1, 2 * D), full),
            pl.BlockSpec((2 * D, D), full),
            pl.BlockSpec((1, D), full),
            pl.BlockSpec((1, D), full),
            pl.BlockSpec((1, D), full),
        ],
        out_specs=pl.BlockSpec((MLP_R, D), row),
        out_shape=jax.ShapeDtypeStruct((N, D), jnp.float32),
    )(scale, h, a0, a1, w1, b1, w2, b2, gamma, beta)


POOL_R = 1000
_POOL_NB = N // POOL_R
T_OUT = 128


def _pool_body(batch_ref, h_ref, pw_ref, pb_ref, o_ref, acc_ref, cnt_ref):
    i = pl.program_id(0)

    @pl.when(i == 0)
    def _():
        acc_ref[...] = jnp.zeros_like(acc_ref)
        cnt_ref[...] = jnp.zeros_like(cnt_ref)

    bb = batch_ref[...]
    iot = lax.broadcasted_iota(jnp.int32, (POOL_R, G), 1)
    oh = (iot == bb).astype(jnp.float32)
    acc_ref[...] += lax.dot_general(
        oh, h_ref[...], (((0,), (0,)), ((), ())),
        preferred_element_type=jnp.float32)
    cnt_ref[...] += lax.dot_general(
        oh, jnp.ones((POOL_R, 1), jnp.float32), (((0,), (0,)), ((), ())),
        preferred_element_type=jnp.float32)

    @pl.when(i == _POOL_NB - 1)
    def _():
        hg = acc_ref[...] / jnp.maximum(cnt_ref[...], 1.0)
        o_ref[...] = jnp.dot(hg, pw_ref[...],
                             preferred_element_type=jnp.float32) + pb_ref[...]


def _pool_head(batch_f, h, pred_W, pred_b):
    row = lambda i: (i, 0)
    full = lambda i: (0, 0)
    return pl.pallas_call(
        _pool_body,
        grid=(_POOL_NB,),
        in_specs=[
            pl.BlockSpec((POOL_R, 1), row),
            pl.BlockSpec((POOL_R, D), row),
            pl.BlockSpec((D, T_OUT), full),
            pl.BlockSpec((1, T_OUT), full),
        ],
        out_specs=pl.BlockSpec((G, T_OUT), full),
        out_shape=jax.ShapeDtypeStruct((G, T_OUT), jnp.float32),
        scratch_shapes=[
            pltpu.VMEM((G, D), jnp.float32),
            pltpu.VMEM((G, 1), jnp.float32),
        ],
    )(batch_f, h, pred_W, pred_b)


def kernel(x, edge_index, edge_attr, batch, mlp_W1, mlp_b1, mlp_W2, mlp_b2,
           eps, gamma, beta, pred_W, pred_b):
    src = edge_index[0]
    dst = edge_index[1]
    h = x
    num_layers = mlp_W1.shape[0]
    for i in range(num_layers):
        agg = _gin_agg(h, src, dst, edge_attr)
        scale = jnp.full((1, D), 1.0, jnp.float32) * (1.0 + eps[i])
        h = _mlp_layer(scale, h, agg[0], agg[1], mlp_W1[i],
                       mlp_b1[i].reshape(1, -1), mlp_W2[i],
                       mlp_b2[i].reshape(1, -1), gamma[i].reshape(1, -1),
                       beta[i].reshape(1, -1), relu_out=(i != num_layers - 1))
    return _pool_head(batch.reshape(N, 1), h, pred_W, pred_b.reshape(1, -1))
